# baseline (speedup 1.0000x reference)
import functools

import jax
import jax.numpy as jnp
from jax import lax
from jax.experimental import pallas as pl
from jax.experimental.pallas import tpu as pltpu
from jax.experimental.pallas import tpu_sc as plsc

N_ROWS = 65536
N_COLS = 60
N_DATA = 30
N_SEG = 31
N_NEW = 33
LANES = 16

NUM_CORES = 2
NUM_SUBCORES = 16
NW = NUM_CORES * NUM_SUBCORES
ROWS_PER_W = N_ROWS // NW
CHUNK = 128
GROUPS = CHUNK // LANES
NCHUNKS = ROWS_PER_W // CHUNK
NBUF = 2


NSTREAM = 2


def _interp_body(x_hbm, out_hbm, x_v0, x_v1, out_v0, out_v1,
                 hist0, hist1, m_buf0, m_buf1, b_buf0, b_buf1,
                 sem_in, sem_out):
    x_v = [x_v0, x_v1]
    out_v = [out_v0, out_v1]
    hists = [hist0, hist1]
    m_bufs = [m_buf0, m_buf1]
    b_bufs = [b_buf0, b_buf1]
    wid = lax.axis_index("s") * NUM_CORES + lax.axis_index("c")
    lane = lax.iota(jnp.int32, LANES)
    row0_w = wid * ROWS_PER_W

    zero_i = jnp.zeros((LANES,), jnp.int32)
    zero_f = jnp.zeros((LANES,), jnp.float32)
    ones = jnp.ones((LANES,), jnp.int32)

    def process_groups(gp, _, xc, oc):
        st = []
        for s in range(NSTREAM):
            g = gp * NSTREAM + s
            st.append(dict(
                h=hists[s], mb=m_bufs[s], bb=b_bufs[s],
                rb60=(g * LANES + lane) * N_COLS,
                rb33=(g * LANES + lane) * N_NEW,
                px=zero_f, py=zero_f, run=zero_i))

        for k in range(N_NEW):
            for t_ in st:
                t_['h'][pl.ds(k * LANES, LANES)] = zero_i

        for j in range(N_DATA):
            for t_ in st:
                t_['cx'] = plsc.load_gather(xc, [t_['rb60'] + j])
                t_['cy'] = plsc.load_gather(xc, [t_['rb60'] + (N_DATA + j)])
            for t_ in st:
                t = t_['cx'] * 32.0
                ti = t.astype(jnp.int32)
                c = ti + (t > ti.astype(jnp.float32)).astype(jnp.int32)
                plsc.addupdate_scatter(t_['h'], [c * LANES + lane], ones)
                m = (t_['cy'] - t_['py']) / (t_['cx'] - t_['px'])
                b = t_['py'] - m * t_['px']
                t_['mb'][pl.ds(j * LANES, LANES)] = m
                t_['bb'][pl.ds(j * LANES, LANES)] = b
                t_['px'], t_['py'] = t_['cx'], t_['cy']
        for t_ in st:
            m = (1.0 - t_['py']) / (1.0 - t_['px'])
            b = t_['py'] - m * t_['px']
            t_['mb'][pl.ds(N_DATA * LANES, LANES)] = m
            t_['bb'][pl.ds(N_DATA * LANES, LANES)] = b

        for k in range(N_NEW):
            for t_ in st:
                t_['run'] = t_['run'] + t_['h'][pl.ds(k * LANES, LANES)]
                idx = jnp.minimum(t_['run'], N_SEG - 1)
                gi = idx * LANES + lane
                ms = plsc.load_gather(t_['mb'], [gi])
                bs = plsc.load_gather(t_['bb'], [gi])
                res = ms * (k * 0.03125) + bs
                plsc.store_scatter(oc, [t_['rb33'] + k], res)
        return 0

    for b in range(NBUF):
        pltpu.async_copy(
            x_hbm.at[pl.ds((row0_w + b * CHUNK) * N_COLS, CHUNK * N_COLS)],
            x_v[b], sem_in.at[b])

    def chunk_round(c2, _):
        for b in range(NBUF):
            c = c2 * NBUF + b
            row0 = row0_w + c * CHUNK
            pltpu.make_async_copy(
                x_hbm.at[pl.ds(row0 * N_COLS, CHUNK * N_COLS)], x_v[b],
                sem_in.at[b]).wait()

            @pl.when(c >= NBUF)
            def _():
                pltpu.make_async_copy(
                    out_v[b],
                    out_hbm.at[pl.ds((row0 - NBUF * CHUNK) * N_NEW,
                                     CHUNK * N_NEW)],
                    sem_out.at[b]).wait()

            lax.fori_loop(
                0, GROUPS // NSTREAM,
                functools.partial(process_groups, xc=x_v[b],
                                  oc=out_v[b]),
                0)

            @pl.when(c + NBUF < NCHUNKS)
            def _():
                pltpu.async_copy(
                    x_hbm.at[pl.ds((row0 + NBUF * CHUNK) * N_COLS,
                                   CHUNK * N_COLS)],
                    x_v[b], sem_in.at[b])

            pltpu.async_copy(
                out_v[b], out_hbm.at[pl.ds(row0 * N_NEW, CHUNK * N_NEW)],
                sem_out.at[b])
        return 0

    lax.fori_loop(0, NCHUNKS // NBUF, chunk_round, 0)

    for b in range(NBUF):
        row0 = row0_w + (NCHUNKS - NBUF + b) * CHUNK
        pltpu.make_async_copy(
            out_v[b], out_hbm.at[pl.ds(row0 * N_NEW, CHUNK * N_NEW)],
            sem_out.at[b]).wait()


def _build():
    mesh = plsc.VectorSubcoreMesh(core_axis_name="c", subcore_axis_name="s")
    return pl.kernel(
        _interp_body,
        mesh=mesh,
        compiler_params=pltpu.CompilerParams(needs_layout_passes=False),
        out_type=jax.ShapeDtypeStruct((N_ROWS * N_NEW,), jnp.float32),
        scratch_types=[
            pltpu.VMEM((CHUNK * N_COLS,), jnp.float32),
            pltpu.VMEM((CHUNK * N_COLS,), jnp.float32),
            pltpu.VMEM((CHUNK * N_NEW,), jnp.float32),
            pltpu.VMEM((CHUNK * N_NEW,), jnp.float32),
            pltpu.VMEM((N_NEW * LANES,), jnp.int32),
            pltpu.VMEM((N_NEW * LANES,), jnp.int32),
            pltpu.VMEM((N_SEG * LANES,), jnp.float32),
            pltpu.VMEM((N_SEG * LANES,), jnp.float32),
            pltpu.VMEM((N_SEG * LANES,), jnp.float32),
            pltpu.VMEM((N_SEG * LANES,), jnp.float32),
            pltpu.SemaphoreType.DMA((NBUF,)),
            pltpu.SemaphoreType.DMA((NBUF,)),
        ],
    )


@jax.jit
def kernel(X):
    flat = _build()(X.reshape(N_ROWS * N_COLS))
    return flat.reshape(N_ROWS, N_NEW)


# speedup vs baseline: 1.8373x; 1.8373x over previous
import functools

import jax
import jax.numpy as jnp
from jax import lax
from jax.experimental import pallas as pl
from jax.experimental.pallas import tpu as pltpu
from jax.experimental.pallas import tpu_sc as plsc

N_ROWS = 65536
N_COLS = 60
N_DATA = 30
N_SEG = 31
N_NEW = 33
LANES = 16

NUM_CORES = 2
NUM_SUBCORES = 16
NW = NUM_CORES * NUM_SUBCORES
ROWS_PER_W = N_ROWS // NW
CHUNK = 128
GROUPS = CHUNK // LANES
NCHUNKS = ROWS_PER_W // CHUNK
NBUF = 2
PRE = 2


def _interp_body(x_hbm, out_hbm, x_v0, x_v1, out_v0, out_v1,
                 hist, m_buf, b_buf, sem_in, sem_out):
    x_v = [x_v0, x_v1]
    out_v = [out_v0, out_v1]
    wid = lax.axis_index("s") * NUM_CORES + lax.axis_index("c")
    lane = lax.iota(jnp.int32, LANES)
    row0_w = wid * ROWS_PER_W

    zero_i = jnp.zeros((LANES,), jnp.int32)
    zero_f = jnp.zeros((LANES,), jnp.float32)
    ones = jnp.ones((LANES,), jnp.int32)

    def process_group(g, _, xc, oc):
        rb60 = (g * LANES + lane) * N_COLS
        rb33 = (g * LANES + lane) * N_NEW

        xs, ys = {}, {}

        def loadx(j):
            xs[j] = plsc.load_gather(xc, [rb60 + j])

        def loady(j):
            ys[j] = plsc.load_gather(xc, [rb60 + (N_DATA + j)])

        for j in range(PRE):
            loadx(j)
            loady(j)
        for k in range(N_NEW):
            hist[pl.ds(k * LANES, LANES)] = zero_i

        px, py = zero_f, zero_f
        for j in range(N_DATA):
            if j + PRE < N_DATA:
                loadx(j + PRE)
                loady(j + PRE)
            cx, cy = xs.pop(j), ys.pop(j)
            t = cx * 32.0
            ti = t.astype(jnp.int32)
            c = ti + (t > ti.astype(jnp.float32)).astype(jnp.int32)
            plsc.addupdate_scatter(hist, [c * LANES + lane], ones)
            m = (cy - py) / (cx - px)
            b = py - m * px
            m_buf[pl.ds(j * LANES, LANES)] = m
            b_buf[pl.ds(j * LANES, LANES)] = b
            px, py = cx, cy
        m = (1.0 - py) / (1.0 - px)
        b = py - m * px
        m_buf[pl.ds(N_DATA * LANES, LANES)] = m
        b_buf[pl.ds(N_DATA * LANES, LANES)] = b

        hs = {}

        def loadh(k):
            hs[k] = hist[pl.ds(k * LANES, LANES)]

        for k in range(PRE):
            loadh(k)
        run = zero_i
        for k in range(N_NEW):
            if k + PRE < N_NEW:
                loadh(k + PRE)
            run = run + hs.pop(k)
            idx = jnp.minimum(run, N_SEG - 1)
            gi = idx * LANES + lane
            ms = plsc.load_gather(m_buf, [gi])
            bs = plsc.load_gather(b_buf, [gi])
            res = ms * (k * 0.03125) + bs
            plsc.store_scatter(oc, [rb33 + k], res)
        return 0

    for b in range(NBUF):
        pltpu.async_copy(
            x_hbm.at[pl.ds((row0_w + b * CHUNK) * N_COLS, CHUNK * N_COLS)],
            x_v[b], sem_in.at[b])

    def chunk_round(c2, _):
        for b in range(NBUF):
            c = c2 * NBUF + b
            row0 = row0_w + c * CHUNK
            pltpu.make_async_copy(
                x_hbm.at[pl.ds(row0 * N_COLS, CHUNK * N_COLS)], x_v[b],
                sem_in.at[b]).wait()

            @pl.when(c >= NBUF)
            def _():
                pltpu.make_async_copy(
                    out_v[b],
                    out_hbm.at[pl.ds((row0 - NBUF * CHUNK) * N_NEW,
                                     CHUNK * N_NEW)],
                    sem_out.at[b]).wait()

            lax.fori_loop(
                0, GROUPS,
                functools.partial(process_group, xc=x_v[b], oc=out_v[b]),
                0)

            @pl.when(c + NBUF < NCHUNKS)
            def _():
                pltpu.async_copy(
                    x_hbm.at[pl.ds((row0 + NBUF * CHUNK) * N_COLS,
                                   CHUNK * N_COLS)],
                    x_v[b], sem_in.at[b])

            pltpu.async_copy(
                out_v[b], out_hbm.at[pl.ds(row0 * N_NEW, CHUNK * N_NEW)],
                sem_out.at[b])
        return 0

    lax.fori_loop(0, NCHUNKS // NBUF, chunk_round, 0)

    for b in range(NBUF):
        row0 = row0_w + (NCHUNKS - NBUF + b) * CHUNK
        pltpu.make_async_copy(
            out_v[b], out_hbm.at[pl.ds(row0 * N_NEW, CHUNK * N_NEW)],
            sem_out.at[b]).wait()


def _build():
    mesh = plsc.VectorSubcoreMesh(core_axis_name="c", subcore_axis_name="s")
    return pl.kernel(
        _interp_body,
        mesh=mesh,
        compiler_params=pltpu.CompilerParams(needs_layout_passes=False),
        out_type=jax.ShapeDtypeStruct((N_ROWS * N_NEW,), jnp.float32),
        scratch_types=[
            pltpu.VMEM((CHUNK * N_COLS,), jnp.float32),
            pltpu.VMEM((CHUNK * N_COLS,), jnp.float32),
            pltpu.VMEM((CHUNK * N_NEW,), jnp.float32),
            pltpu.VMEM((CHUNK * N_NEW,), jnp.float32),
            pltpu.VMEM((N_NEW * LANES,), jnp.int32),
            pltpu.VMEM((N_SEG * LANES,), jnp.float32),
            pltpu.VMEM((N_SEG * LANES,), jnp.float32),
            pltpu.SemaphoreType.DMA((NBUF,)),
            pltpu.SemaphoreType.DMA((NBUF,)),
        ],
    )


@jax.jit
def kernel(X):
    flat = _build()(X.reshape(N_ROWS * N_COLS))
    return flat.reshape(N_ROWS, N_NEW)
